# baseline (speedup 1.0000x reference)
import functools

import jax
import jax.numpy as jnp
from jax import lax
from jax.experimental import pallas as pl
from jax.experimental.pallas import tpu as pltpu
from jax.experimental.pallas import tpu_sc as plsc

_S = 19
_DH = 64
_D = 2 * _DH
_NC, _NS, _L = 2, 16, 16
_NW = _NC * _NS
_B, _LN = 4096, 200
_N = _B * _LN
_PW = _N // _NW
_CHT = 64
_CHI = 2 * _CHT
_NCH = _PW // _CHT
_NBUF = 4
_SEG = 1600
_NSEG = _PW // _SEG
_CPS = _SEG // _CHT


def _sc_body(co_hbm, row_hbm, col_hbm, out_hbm,
             co_v, idx_v, tab_v, rows_v, gsem, ssem):
  wid = lax.axis_index("s") * _NC + lax.axis_index("c")
  base0 = wid * _PW

  def load_tab():
    pltpu.sync_copy(row_hbm, tab_v.at[pl.ds(0, _S)])
    pltpu.sync_copy(col_hbm, tab_v.at[pl.ds(_S, _S)])
  pl.when(lax.axis_index("s") == 0)(load_tab)
  plsc.subcore_barrier()

  offs = (lax.iota(jnp.int32, _L) % 2) * _S

  def seg(s, _):
    pltpu.sync_copy(co_hbm.at[pl.ds(2 * (base0 + s * _SEG), 2 * _SEG)], co_v)
    def per_chunk(t, _):
      def per_vec(j, _):
        pos = t * _CHI + j * _L
        iv = jnp.maximum(co_v[pl.ds(pos, _L)], 0) + offs
        idx_v[s * _CPS + t, pl.ds(j * _L, _L)] = iv
        return 0
      lax.fori_loop(0, _CHI // _L, per_vec, 0)
      return 0
    lax.fori_loop(0, _CPS, per_chunk, 0)
    return 0
  lax.fori_loop(0, _NSEG, seg, 0)

  def out_at(k):
    return out_hbm.at[pl.ds(2 * base0 + k * _CHI, _CHI)]

  def rnd(i, _):
    descs = []
    for b in range(_NBUF):
      k = i * _NBUF + b
      def wait_prev(b=b, k=k):
        pltpu.make_async_copy(rows_v.at[b], out_at(k - _NBUF), ssem.at[b]).wait()
      pl.when(i > 0)(wait_prev)
      descs.append(
          pltpu.async_copy(tab_v.at[idx_v.at[k]], rows_v.at[b], gsem.at[b]))
    for b in range(_NBUF):
      k = i * _NBUF + b
      descs[b].wait()
      pltpu.async_copy(rows_v.at[b], out_at(k), ssem.at[b])
    return 0
  lax.fori_loop(0, _NCH // _NBUF, rnd, 0)
  for b in range(_NBUF):
    k = _NCH - _NBUF + b
    pltpu.make_async_copy(rows_v.at[b], out_at(k), ssem.at[b]).wait()


_sc_gather = functools.partial(
    pl.kernel,
    out_type=jax.ShapeDtypeStruct((2 * _N, _DH), jnp.float32),
    mesh=plsc.VectorSubcoreMesh(
        core_axis_name="c", subcore_axis_name="s",
        num_cores=_NC, num_subcores=_NS),
    compiler_params=pltpu.CompilerParams(use_tc_tiling_on_sc=False),
    scratch_types=[
        pltpu.VMEM((2 * _SEG,), jnp.int32),
        pltpu.VMEM((_NCH, _CHI), jnp.int32),
        pltpu.VMEM_SHARED((2 * _S, _DH), jnp.float32),
        pltpu.VMEM((_NBUF, _CHI, _DH), jnp.float32),
        pltpu.SemaphoreType.DMA((_NBUF,)),
        pltpu.SemaphoreType.DMA((_NBUF,)),
    ],
)(_sc_body)


def kernel(coords, row_emb, col_emb):
  co = coords.reshape(2 * _N)
  out = _sc_gather(co, row_emb, col_emb)
  return out.reshape(_B, _LN, _D)


# speedup vs baseline: 5.7012x; 5.7012x over previous
import functools

import jax
import jax.numpy as jnp
from jax import lax
from jax.experimental import pallas as pl
from jax.experimental.pallas import tpu as pltpu
from jax.experimental.pallas import tpu_sc as plsc

_S = 19
_D = 128
_DH = 64
_NC, _NS, _L = 2, 16, 16
_NW = _NC * _NS
_B, _LN = 4096, 200
_N = _B * _LN
_PW = _N // _NW
_CH = 64
_NCH = _PW // _CH
_NBUF = 8
_SEG = 2560
_NSEG = _PW // _SEG
_CPS = _SEG // _CH
_RPS = _CPS // _NBUF
_VPC = _CH // _L


def _table_body(row_ref, col_ref, out_ref):
  r = jnp.broadcast_to(row_ref[...][:, None, :], (_S, _S, _DH))
  c = jnp.broadcast_to(col_ref[...][None, :, :], (_S, _S, _DH))
  out_ref[...] = jnp.concatenate([r, c], axis=-1)


def _build_table(row_emb, col_emb):
  return pl.pallas_call(
      _table_body,
      out_shape=jax.ShapeDtypeStruct((_S, _S, _D), jnp.float32),
  )(row_emb, col_emb).reshape(_S * _S, _D)


def _sc_body(r_hbm, c_hbm, tab_hbm, out_hbm,
             r2_v, c2_v, idx_v, tab_v, rows_v, gsem, ssem, csem):
  wid = lax.axis_index("s") * _NC + lax.axis_index("c")
  base0 = wid * _PW

  def load_tab():
    pltpu.sync_copy(tab_hbm, tab_v)
  pl.when(lax.axis_index("s") == 0)(load_tab)
  plsc.subcore_barrier()

  def seg_slice(hbm, s):
    return hbm.at[pl.ds(base0 + s * _SEG, _SEG)]

  def idx_vec(s, t, j):
    pos = t * _CH + j * _L
    buf = lax.rem(s, 2)
    iv = (jnp.maximum(r2_v[buf, pl.ds(pos, _L)], 0) * _S
          + jnp.maximum(c2_v[buf, pl.ds(pos, _L)], 0))
    idx_v[s * _CPS + t, pl.ds(j * _L, _L)] = iv

  pltpu.sync_copy(seg_slice(r_hbm, 0), r2_v.at[0])
  pltpu.sync_copy(seg_slice(c_hbm, 0), c2_v.at[0])
  def pro_chunk(t, _):
    for j in range(_VPC):
      idx_vec(0, t, j)
    return 0
  lax.fori_loop(0, _CPS, pro_chunk, 0)
  if _NSEG > 1:
    pltpu.async_copy(seg_slice(r_hbm, 1), r2_v.at[1], csem.at[0])
    pltpu.async_copy(seg_slice(c_hbm, 1), c2_v.at[1], csem.at[1])

  def out_at(k):
    return out_hbm.at[pl.ds(base0 + k * _CH, _CH)]

  def seg_loop(s, _):
    def q_loop(q, _):
      g = s * _RPS + q
      descs = []
      for b in range(_NBUF):
        k = g * _NBUF + b
        def wait_prev(b=b, k=k):
          pltpu.make_async_copy(rows_v.at[b], out_at(k - _NBUF),
                                ssem.at[b]).wait()
        pl.when(g > 0)(wait_prev)
        descs.append(
            pltpu.async_copy(tab_v.at[idx_v.at[k]], rows_v.at[b],
                             gsem.at[b]))

      def coords_mgmt():
        def wait_next():
          pltpu.make_async_copy(seg_slice(r_hbm, 0), r2_v.at[0],
                                csem.at[0]).wait()
          pltpu.make_async_copy(seg_slice(c_hbm, 0), c2_v.at[0],
                                csem.at[1]).wait()
        pl.when(s + 1 < _NSEG)(wait_next)
        def issue_next2():
          nb = lax.rem(s, 2)
          pltpu.async_copy(seg_slice(r_hbm, s + 2), r2_v.at[nb], csem.at[0])
          pltpu.async_copy(seg_slice(c_hbm, s + 2), c2_v.at[nb], csem.at[1])
        pl.when(s + 2 < _NSEG)(issue_next2)
      pl.when(q == 0)(coords_mgmt)

      def next_idx():
        for u in range(_NBUF):
          for j in range(_VPC):
            idx_vec(s + 1, q * _NBUF + u, j)
      pl.when(s + 1 < _NSEG)(next_idx)

      for b in range(_NBUF):
        k = g * _NBUF + b
        descs[b].wait()
        pltpu.async_copy(rows_v.at[b], out_at(k), ssem.at[b])
      return 0
    lax.fori_loop(0, _RPS, q_loop, 0)
    return 0
  lax.fori_loop(0, _NSEG, seg_loop, 0)

  for b in range(_NBUF):
    k = _NCH - _NBUF + b
    pltpu.make_async_copy(rows_v.at[b], out_at(k), ssem.at[b]).wait()


_sc_gather = functools.partial(
    pl.kernel,
    out_type=jax.ShapeDtypeStruct((_N, _D), jnp.float32),
    mesh=plsc.VectorSubcoreMesh(
        core_axis_name="c", subcore_axis_name="s",
        num_cores=_NC, num_subcores=_NS),
    scratch_types=[
        pltpu.VMEM((2, _SEG), jnp.int32),
        pltpu.VMEM((2, _SEG), jnp.int32),
        pltpu.VMEM((_NCH, _CH), jnp.int32),
        pltpu.VMEM_SHARED((_S * _S, _D), jnp.float32),
        pltpu.VMEM((_NBUF, _CH, _D), jnp.float32),
        pltpu.SemaphoreType.DMA((_NBUF,)),
        pltpu.SemaphoreType.DMA((_NBUF,)),
        pltpu.SemaphoreType.DMA((2,)),
    ],
)(_sc_body)


def kernel(coords, row_emb, col_emb):
  table = _build_table(row_emb, col_emb)
  r = coords[..., 0].reshape(_N)
  c = coords[..., 1].reshape(_N)
  out = _sc_gather(r, c, table)
  return out.reshape(_B, _LN, _D)


# speedup vs baseline: 6.1135x; 1.0723x over previous
import functools

import jax
import jax.numpy as jnp
from jax import lax
from jax.experimental import pallas as pl
from jax.experimental.pallas import tpu as pltpu
from jax.experimental.pallas import tpu_sc as plsc

_S = 19
_D = 128
_DH = 64
_NC, _NS, _L = 2, 16, 16
_NW = _NC * _NS
_B, _LN = 4096, 200
_N = _B * _LN
_PW = _N // _NW
_CH = 64
_NCH = _PW // _CH
_NBUF = 8
_SEG = 2560
_NSEG = _PW // _SEG
_CPS = _SEG // _CH
_RPS = _CPS // _NBUF
_VPC = _CH // _L


def _table_body(row_ref, col_ref, out_ref):
  r = jnp.broadcast_to(row_ref[...][:, None, :], (_S, _S, _DH))
  c = jnp.broadcast_to(col_ref[...][None, :, :], (_S, _S, _DH))
  out_ref[...] = jnp.concatenate([r, c], axis=-1)


def _build_table(row_emb, col_emb):
  return pl.pallas_call(
      _table_body,
      out_shape=jax.ShapeDtypeStruct((_S, _S, _D), jnp.float32),
  )(row_emb, col_emb).reshape(_S * _S, _D)


def _sc_body(rc_hbm, tab_hbm, out_hbm,
             rc2_v, idx_v, tab_v, rows_v, gsem, ssem, csem):
  wid = lax.axis_index("s") * _NC + lax.axis_index("c")
  base0 = wid * _PW

  def load_tab():
    pltpu.sync_copy(tab_hbm, tab_v)
  pl.when(lax.axis_index("s") == 0)(load_tab)
  plsc.subcore_barrier()

  def seg_slice(hbm, s):
    return hbm.at[pl.ds(base0 + s * _SEG, _SEG)]

  def idx_vec(s, t, j):
    pos = t * _CH + j * _L
    buf = lax.rem(s, 2)
    v = rc2_v[buf, pl.ds(pos, _L)]
    r = jnp.bitwise_and(v, 0xFFFF)
    c = jnp.right_shift(v, 16)
    iv = jnp.maximum(r, 0) * _S + jnp.maximum(c, 0)
    idx_v[s * _CPS + t, pl.ds(j * _L, _L)] = iv

  pltpu.sync_copy(seg_slice(rc_hbm, 0), rc2_v.at[0])
  def pro_chunk(t, _):
    for j in range(_VPC):
      idx_vec(0, t, j)
    return 0
  lax.fori_loop(0, _CPS, pro_chunk, 0)
  if _NSEG > 1:
    pltpu.async_copy(seg_slice(rc_hbm, 1), rc2_v.at[1], csem.at[0])

  def out_at(k):
    return out_hbm.at[pl.ds(base0 + k * _CH, _CH)]

  def seg_loop(s, _):
    def q_loop(q, _):
      g = s * _RPS + q
      descs = []
      for b in range(_NBUF):
        k = g * _NBUF + b
        def wait_prev(b=b, k=k):
          pltpu.make_async_copy(rows_v.at[b], out_at(k - _NBUF),
                                ssem.at[b]).wait()
        pl.when(g > 0)(wait_prev)
        descs.append(
            pltpu.async_copy(tab_v.at[idx_v.at[k]], rows_v.at[b],
                             gsem.at[b]))

      def coords_mgmt():
        def wait_next():
          pltpu.make_async_copy(seg_slice(rc_hbm, 0), rc2_v.at[0],
                                csem.at[0]).wait()
        pl.when(s + 1 < _NSEG)(wait_next)
        def issue_next2():
          nb = lax.rem(s, 2)
          pltpu.async_copy(seg_slice(rc_hbm, s + 2), rc2_v.at[nb], csem.at[0])
        pl.when(s + 2 < _NSEG)(issue_next2)
      pl.when(q == 0)(coords_mgmt)

      def next_idx():
        for u in range(_NBUF):
          for j in range(_VPC):
            idx_vec(s + 1, q * _NBUF + u, j)
      pl.when(s + 1 < _NSEG)(next_idx)

      for b in range(_NBUF):
        k = g * _NBUF + b
        descs[b].wait()
        pltpu.async_copy(rows_v.at[b], out_at(k), ssem.at[b])
      return 0
    lax.fori_loop(0, _RPS, q_loop, 0)
    return 0
  lax.fori_loop(0, _NSEG, seg_loop, 0)

  for b in range(_NBUF):
    k = _NCH - _NBUF + b
    pltpu.make_async_copy(rows_v.at[b], out_at(k), ssem.at[b]).wait()


_sc_gather = functools.partial(
    pl.kernel,
    out_type=jax.ShapeDtypeStruct((_N, _D), jnp.float32),
    mesh=plsc.VectorSubcoreMesh(
        core_axis_name="c", subcore_axis_name="s",
        num_cores=_NC, num_subcores=_NS),
    scratch_types=[
        pltpu.VMEM((2, _SEG), jnp.int32),
        pltpu.VMEM((_NCH, _CH), jnp.int32),
        pltpu.VMEM_SHARED((_S * _S, _D), jnp.float32),
        pltpu.VMEM((_NBUF, _CH, _D), jnp.float32),
        pltpu.SemaphoreType.DMA((_NBUF,)),
        pltpu.SemaphoreType.DMA((_NBUF,)),
        pltpu.SemaphoreType.DMA((1,)),
    ],
)(_sc_body)


def kernel(coords, row_emb, col_emb):
  table = _build_table(row_emb, col_emb)
  rc = jnp.bitwise_or(coords[..., 0],
                      jnp.left_shift(coords[..., 1], 16)).reshape(_N)
  out = _sc_gather(rc, table)
  return out.reshape(_B, _LN, _D)
